# Initial kernel scaffold; baseline (speedup 1.0000x reference)
import jax
import jax.numpy as jnp
from jax.experimental import pallas as pl


def kernel(input_ids, pos_table, type_table, ln_weight, ln_bias):
    raise NotImplementedError("write your pallas kernel here")


# speedup vs baseline: 6.4697x; 6.4697x over previous
import jax
import jax.numpy as jnp
from jax.experimental import pallas as pl

EPS = 1e-12
TILE = 256


def _ln_kernel(pos_ref, type_ref, w_ref, b_ref, out_ref):
    x = pos_ref[...] + type_ref[0, :][None, :]
    mean = jnp.mean(x, axis=1, keepdims=True)
    xc = x - mean
    var = jnp.mean(xc * xc, axis=1, keepdims=True)
    y = xc * jax.lax.rsqrt(var + EPS)
    y = y * w_ref[0, :][None, :] + b_ref[0, :][None, :]
    out_ref[...] = jnp.broadcast_to(y[None, :, :], out_ref.shape)


def kernel(input_ids, pos_table, type_table, ln_weight, ln_bias):
    b, s = input_ids.shape
    h = pos_table.shape[1]
    w2 = ln_weight.reshape(1, h)
    b2 = ln_bias.reshape(1, h)
    grid = (s // TILE,)
    out = pl.pallas_call(
        _ln_kernel,
        grid=grid,
        in_specs=[
            pl.BlockSpec((TILE, h), lambda i: (i, 0)),
            pl.BlockSpec(type_table.shape, lambda i: (0, 0)),
            pl.BlockSpec((1, h), lambda i: (0, 0)),
            pl.BlockSpec((1, h), lambda i: (0, 0)),
        ],
        out_specs=pl.BlockSpec((b, TILE, h), lambda i: (0, i, 0)),
        out_shape=jax.ShapeDtypeStruct((b, s, h), jnp.float32),
    )(pos_table[:s], type_table, w2, b2)
    return out
